# baseline (device time: 101035 ns/iter reference)
import jax
import jax.numpy as jnp
import numpy as np
from jax import lax
from jax.experimental import pallas as pl
from jax.experimental.pallas import tpu as pltpu

N_DEV = 32
HF = 16
HB = 15


def _logical_coords():
    order = []
    for z in range(4):
        for yi in range(4):
            row = [(x, yi, z) for x in range(2)]
            if yi % 2:
                row = row[::-1]
            order.extend(row)
    return order


def _hamiltonian_cycle():
    path0 = []
    for zi in range(4):
        ys = range(4) if zi % 2 == 0 else range(3, -1, -1)
        for y in ys:
            path0.append((0, y, z := zi))
    path1 = [(1, y, z) for (_, y, z) in reversed(path0)]
    return path0 + path1


_COORD_TO_LOGICAL = {c: i for i, c in enumerate(_logical_coords())}
CYCLE = np.array([_COORD_TO_LOGICAL[c] for c in _hamiltonian_cycle()],
                 dtype=np.int32)
POS = np.empty(N_DEV, dtype=np.int32)
POS[CYCLE] = np.arange(N_DEV, dtype=np.int32)


def kernel(x, w_mat, scale_x, scale_w):
    m_per, k = x.shape
    _, n_per = w_mat.shape
    m_glob = N_DEV * m_per

    def body(x_ref, w_ref, sx_ref, sw_ref, cyc_ref, pos_ref, out_ref,
             gather_ref, w8_ref, fsend, frecv, bsend, brecv, dummy):
        my = lax.axis_index("i")
        r = pos_ref[my]
        right = cyc_ref[lax.rem(r + 1, N_DEV)]
        left = cyc_ref[lax.rem(r - 1 + N_DEV, N_DEV)]

        gather_ref[pl.ds(my * m_per, m_per), :] = (
            x_ref[...].astype(jnp.float8_e4m3fn))
        w8_ref[...] = w_ref[...].astype(jnp.float8_e4m3fn)

        barrier_sem = pltpu.get_barrier_semaphore()
        for nbr in (left, right):
            pl.semaphore_signal(
                barrier_sem, inc=1,
                device_id=(nbr,), device_id_type=pl.DeviceIdType.MESH,
            )
        pl.semaphore_wait(barrier_sem, 2)

        scale = sx_ref[0] * sw_ref[0]

        def compute(chunk_ref, origin):
            acc = jax.lax.dot_general(
                chunk_ref[...], w8_ref[...],
                dimension_numbers=(((1,), (0,)), ((), ())),
                preferred_element_type=jnp.float32,
            )
            y = acc * scale
            out_ref[pl.ds(origin * m_per, m_per), :] = (
                y * (1.0 / (1.0 + jnp.exp(-y)))
            )

        m_sub = m_per // 2

        def rows(origin):
            return pl.ds(origin * m_per, m_per)

        def sub_rows(origin, j):
            return pl.ds(origin * m_per + j * m_sub, m_sub)

        def make(origin, j, send_sem, recv_sem, dev):
            sl = gather_ref.at[sub_rows(origin, j), :]
            return pltpu.make_async_remote_copy(
                src_ref=sl,
                dst_ref=sl,
                send_sem=send_sem,
                recv_sem=recv_sem,
                device_id=(dev,),
                device_id_type=pl.DeviceIdType.MESH,
            )

        f_prev = [None, None]
        b_prev = [None, None]
        for j in range(2):
            f_prev[j] = make(my, j, fsend.at[0, j], frecv.at[0, j], right)
            f_prev[j].start()
            b_prev[j] = make(my, j, bsend.at[0, j], brecv.at[0, j], left)
            b_prev[j].start()
        compute(gather_ref.at[rows(my), :], my)

        for h in range(HF):
            o_f = cyc_ref[lax.rem(r - 1 - h + 2 * N_DEV, N_DEV)]
            for j in ((0, 1) if h < HF - 1 else (0,)):
                rcv = make(o_f, j, dummy.at[0], frecv.at[h, j], left)
                rcv.wait_recv()
                nh = h + 1
                if nh < HF - 1 or (nh == HF - 1 and j == 0):
                    snd = make(o_f, j, fsend.at[nh, j],
                               frecv.at[nh, j], right)
                    snd.start()
                    f_prev[j].wait_send()
                    f_prev[j] = snd
            if h < HF - 1:
                compute(gather_ref.at[rows(o_f), :], o_f)

            o_b = cyc_ref[lax.rem(r + 1 + h, N_DEV)]
            for j in ((0, 1) if h < HB else (1,)):
                rcvb = make(o_b, j, dummy.at[0], brecv.at[h, j], right)
                rcvb.wait_recv()
                nh = h + 1
                if nh < HB or (nh == HB and j == 1):
                    sndb = make(o_b, j, bsend.at[nh, j],
                                brecv.at[nh, j], left)
                    sndb.start()
                    b_prev[j].wait_send()
                    b_prev[j] = sndb
            if h < HB:
                compute(gather_ref.at[rows(o_b), :], o_b)
            else:
                compute(gather_ref.at[rows(o_b), :], o_b)

        for j in range(2):
            f_prev[j].wait_send()
            b_prev[j].wait_send()

    return pl.pallas_call(
        body,
        out_shape=jax.ShapeDtypeStruct((m_glob, n_per), jnp.float32),
        in_specs=[
            pl.BlockSpec(memory_space=pltpu.VMEM),
            pl.BlockSpec(memory_space=pltpu.VMEM),
            pl.BlockSpec(memory_space=pltpu.SMEM),
            pl.BlockSpec(memory_space=pltpu.SMEM),
            pl.BlockSpec(memory_space=pltpu.SMEM),
            pl.BlockSpec(memory_space=pltpu.SMEM),
        ],
        out_specs=pl.BlockSpec(memory_space=pltpu.VMEM),
        scratch_shapes=[
            pltpu.VMEM((m_glob, k), jnp.float8_e4m3fn),
            pltpu.VMEM((k, n_per), jnp.float8_e4m3fn),
            pltpu.SemaphoreType.DMA((HF, 2)),
            pltpu.SemaphoreType.DMA((HF, 2)),
            pltpu.SemaphoreType.DMA((HF, 2)),
            pltpu.SemaphoreType.DMA((HF, 2)),
            pltpu.SemaphoreType.DMA((1,)),
        ],
        compiler_params=pltpu.CompilerParams(collective_id=0),
    )(x, w_mat, scale_x, scale_w, jnp.asarray(CYCLE), jnp.asarray(POS))


# device time: 100594 ns/iter; 1.0044x vs baseline; 1.0044x over previous
import jax
import jax.numpy as jnp
import numpy as np
from jax import lax
from jax.experimental import pallas as pl
from jax.experimental.pallas import tpu as pltpu

N_DEV = 32
HF = 16
HB = 15


def _logical_coords():
    order = []
    for z in range(4):
        for yi in range(4):
            row = [(x, yi, z) for x in range(2)]
            if yi % 2:
                row = row[::-1]
            order.extend(row)
    return order


def _hamiltonian_cycle():
    path0 = []
    for zi in range(4):
        ys = range(4) if zi % 2 == 0 else range(3, -1, -1)
        for y in ys:
            path0.append((0, y, z := zi))
    path1 = [(1, y, z) for (_, y, z) in reversed(path0)]
    return path0 + path1


_COORD_TO_LOGICAL = {c: i for i, c in enumerate(_logical_coords())}
CYCLE = np.array([_COORD_TO_LOGICAL[c] for c in _hamiltonian_cycle()],
                 dtype=np.int32)
POS = np.empty(N_DEV, dtype=np.int32)
POS[CYCLE] = np.arange(N_DEV, dtype=np.int32)


def kernel(x, w_mat, scale_x, scale_w):
    m_per, k = x.shape
    _, n_per = w_mat.shape
    m_glob = N_DEV * m_per

    def body(x_ref, w_ref, sx_ref, sw_ref, cyc_ref, pos_ref, out_ref,
             gather_ref, w8_ref, fsend, frecv, bsend, brecv, dummy):
        my = lax.axis_index("i")
        r = pos_ref[my]
        right = cyc_ref[lax.rem(r + 1, N_DEV)]
        left = cyc_ref[lax.rem(r - 1 + N_DEV, N_DEV)]

        gather_ref[pl.ds(my * m_per, m_per), :] = (
            x_ref[...].astype(jnp.float8_e4m3fn))
        w8_ref[...] = w_ref[...].astype(jnp.float8_e4m3fn)

        barrier_sem = pltpu.get_barrier_semaphore()
        for nbr in (left, right):
            pl.semaphore_signal(
                barrier_sem, inc=1,
                device_id=(nbr,), device_id_type=pl.DeviceIdType.MESH,
            )
        pl.semaphore_wait(barrier_sem, 2)

        scale = sx_ref[0] * sw_ref[0]

        def compute(chunk_ref, origin):
            acc = jax.lax.dot_general(
                chunk_ref[...], w8_ref[...],
                dimension_numbers=(((1,), (0,)), ((), ())),
                preferred_element_type=jnp.float32,
            )
            y = acc * scale
            out_ref[pl.ds(origin * m_per, m_per), :] = (
                y * (1.0 / (1.0 + jnp.exp(-y)))
            )

        m_sub = m_per // 2

        def rows(origin):
            return pl.ds(origin * m_per, m_per)

        def sub_rows(origin, j):
            return pl.ds(origin * m_per + j * m_sub, m_sub)

        def make(origin, j, send_sem, recv_sem, dev):
            sl = gather_ref.at[sub_rows(origin, j), :]
            return pltpu.make_async_remote_copy(
                src_ref=sl,
                dst_ref=sl,
                send_sem=send_sem,
                recv_sem=recv_sem,
                device_id=(dev,),
                device_id_type=pl.DeviceIdType.MESH,
            )

        f_prev = [None, None]
        b_prev = [None, None]
        for j in range(2):
            f_prev[j] = make(my, j, fsend.at[0, j], frecv.at[0, j], right)
            f_prev[j].start()
            b_prev[j] = make(my, j, bsend.at[0, j], brecv.at[0, j], left)
            b_prev[j].start()
        compute(gather_ref.at[rows(my), :], my)

        for h in range(HF):
            o_f = cyc_ref[lax.rem(r - 1 - h + 2 * N_DEV, N_DEV)]
            for j in ((0, 1) if h < HF - 1 else (0,)):
                rcv = make(o_f, j, dummy.at[0], frecv.at[h, j], left)
                rcv.wait_recv()
                nh = h + 1
                if nh < HF - 1 or (nh == HF - 1 and j == 0):
                    snd = make(o_f, j, fsend.at[nh, j],
                               frecv.at[nh, j], right)
                    snd.start()
                    f_prev[j].wait_send()
                    f_prev[j] = snd

            o_b = cyc_ref[lax.rem(r + 1 + h, N_DEV)]
            for j in ((0, 1) if h < HB else (1,)):
                rcvb = make(o_b, j, dummy.at[0], brecv.at[h, j], right)
                rcvb.wait_recv()
                nh = h + 1
                if nh < HB or (nh == HB and j == 1):
                    sndb = make(o_b, j, bsend.at[nh, j],
                                brecv.at[nh, j], left)
                    sndb.start()
                    b_prev[j].wait_send()
                    b_prev[j] = sndb

            if h < HF - 1:
                compute(gather_ref.at[rows(o_f), :], o_f)
            compute(gather_ref.at[rows(o_b), :], o_b)

        for j in range(2):
            f_prev[j].wait_send()
            b_prev[j].wait_send()

    return pl.pallas_call(
        body,
        out_shape=jax.ShapeDtypeStruct((m_glob, n_per), jnp.float32),
        in_specs=[
            pl.BlockSpec(memory_space=pltpu.VMEM),
            pl.BlockSpec(memory_space=pltpu.VMEM),
            pl.BlockSpec(memory_space=pltpu.SMEM),
            pl.BlockSpec(memory_space=pltpu.SMEM),
            pl.BlockSpec(memory_space=pltpu.SMEM),
            pl.BlockSpec(memory_space=pltpu.SMEM),
        ],
        out_specs=pl.BlockSpec(memory_space=pltpu.VMEM),
        scratch_shapes=[
            pltpu.VMEM((m_glob, k), jnp.float8_e4m3fn),
            pltpu.VMEM((k, n_per), jnp.float8_e4m3fn),
            pltpu.SemaphoreType.DMA((HF, 2)),
            pltpu.SemaphoreType.DMA((HF, 2)),
            pltpu.SemaphoreType.DMA((HF, 2)),
            pltpu.SemaphoreType.DMA((HF, 2)),
            pltpu.SemaphoreType.DMA((1,)),
        ],
        compiler_params=pltpu.CompilerParams(collective_id=0),
    )(x, w_mat, scale_x, scale_w, jnp.asarray(CYCLE), jnp.asarray(POS))
